# baseline (speedup 1.0000x reference)
import functools

import jax
import jax.numpy as jnp
from jax import lax
from jax.experimental import pallas as pl
from jax.experimental.pallas import tpu as pltpu
from jax.experimental.pallas import tpu_sc as plsc

NEG_V = -1e30
SCORE_T = 0.01
NMS_T = 0.5
MAX_DET = 100
N_BOX = 20000
N_CLS = 8
LANES = 128
D_TAB = 16


def _fd_kernel(scoresT_ref, boxesT_ref,
               out_scores_ref, out_labels_ref, out_idx_ref,
               s_ref, x1_ref, y1_ref, x2_ref, y2_ref, ar_ref, io_ref):
    ones = jnp.ones((N_CLS, 1), jnp.float32)
    x1_ref[:] = ones * boxesT_ref[0:1, :]
    y1_ref[:] = ones * boxesT_ref[1:2, :]
    x2_ref[:] = ones * boxesT_ref[2:3, :]
    y2_ref[:] = ones * boxesT_ref[3:4, :]
    ar_ref[:] = (x2_ref[:] - x1_ref[:]) * (y2_ref[:] - y1_ref[:])
    io_ref[:] = lax.broadcasted_iota(jnp.int32, (N_CLS, N_BOX), 1)

    sc = scoresT_ref[:]
    s0 = jnp.where(sc > SCORE_T, sc, NEG_V)
    s_ref[:] = s0
    idx0 = jnp.argmax(s0, axis=1).reshape(N_CLS, 1)
    val0 = jnp.max(s0, axis=1, keepdims=True)

    lane_iota = lax.broadcasted_iota(jnp.int32, (N_CLS, LANES), 1)

    def nms_body(k, carry):
        val, idx, vals, idxs = carry
        iota_n = io_ref[:]
        onehot = iota_n == idx
        x1 = x1_ref[:]
        y1 = y1_ref[:]
        x2 = x2_ref[:]
        y2 = y2_ref[:]
        bx1 = jnp.max(jnp.where(onehot, x1, NEG_V), axis=1, keepdims=True)
        by1 = jnp.max(jnp.where(onehot, y1, NEG_V), axis=1, keepdims=True)
        bx2 = jnp.max(jnp.where(onehot, x2, NEG_V), axis=1, keepdims=True)
        by2 = jnp.max(jnp.where(onehot, y2, NEG_V), axis=1, keepdims=True)
        ba = (bx2 - bx1) * (by2 - by1)
        xx1 = jnp.maximum(x1, bx1)
        yy1 = jnp.maximum(y1, by1)
        xx2 = jnp.minimum(x2, bx2)
        yy2 = jnp.minimum(y2, by2)
        inter = jnp.maximum(xx2 - xx1, 0.0) * jnp.maximum(yy2 - yy1, 0.0)
        iou = inter / (ar_ref[:] + ba - inter + 1e-9)
        s_new = jnp.where((iou > NMS_T) | onehot, NEG_V, s_ref[:])
        s_ref[:] = s_new
        idx_n = jnp.argmax(s_new, axis=1).reshape(N_CLS, 1)
        val_n = jnp.max(s_new, axis=1, keepdims=True)
        here = lane_iota == k
        vals = jnp.where(here, val, vals)
        idxs = jnp.where(here, idx, idxs)
        return val_n, idx_n, vals, idxs

    vals0 = jnp.full((N_CLS, LANES), NEG_V, jnp.float32)
    idxs0 = jnp.zeros((N_CLS, LANES), jnp.int32)
    _, _, vals, idxs = lax.fori_loop(
        0, MAX_DET, nms_body, (val0, idx0, vals0, idxs0))

    cand = jnp.where(vals > NEG_V / 2, vals, NEG_V)
    c8 = lax.broadcasted_iota(jnp.int32, (N_CLS, 1), 0)
    lane1 = lax.broadcasted_iota(jnp.int32, (1, LANES), 1)

    def merge_body(t, carry):
        ptr, head, head_idx, tval, tlab, tidx = carry
        m = jnp.max(head, axis=(0, 1), keepdims=True)
        cw = jnp.min(jnp.where(head == m, c8, N_CLS), axis=(0, 1),
                     keepdims=True)
        isw = c8 == cw
        oidx = jnp.max(jnp.where(isw, head_idx, -1), axis=(0, 1),
                       keepdims=True)
        here = lane1 == t
        tval = jnp.where(here, m, tval)
        tlab = jnp.where(here, cw, tlab)
        tidx = jnp.where(here, oidx, tidx)
        ptr = ptr + isw.astype(jnp.int32)
        sel = lane_iota == ptr
        nh = jnp.max(jnp.where(sel, cand, NEG_V), axis=1, keepdims=True)
        nhi = jnp.max(jnp.where(sel, idxs, -1), axis=1, keepdims=True)
        head = jnp.where(isw, nh, head)
        head_idx = jnp.where(isw, nhi, head_idx)
        return ptr, head, head_idx, tval, tlab, tidx

    ptr0 = jnp.zeros((N_CLS, 1), jnp.int32)
    head0 = cand[:, 0:1]
    head_idx0 = idxs[:, 0:1]
    tval0 = jnp.full((1, LANES), NEG_V, jnp.float32)
    tlab0 = jnp.zeros((1, LANES), jnp.int32)
    tidx0 = jnp.zeros((1, LANES), jnp.int32)
    _, _, _, tval, tlab, tidx = lax.fori_loop(
        0, MAX_DET, merge_body,
        (ptr0, head0, head_idx0, tval0, tlab0, tidx0))

    valid = tval > NEG_V / 2
    out_scores_ref[:] = jnp.where(valid, tval, -1.0)
    out_labels_ref[:] = jnp.where(valid, tlab, -1)
    out_idx_ref[:] = jnp.where(valid, tidx, jnp.int32(N_BOX))


def _make_sc_gather():
    mesh = plsc.VectorSubcoreMesh(core_axis_name="c", subcore_axis_name="s")

    @functools.partial(
        pl.kernel, mesh=mesh,
        out_type=jax.ShapeDtypeStruct((LANES, D_TAB), jnp.float32),
        scratch_types=[
            pltpu.VMEM((LANES,), jnp.int32),
            pltpu.VMEM((LANES, D_TAB), jnp.float32),
            pltpu.SemaphoreType.DMA,
        ],
        compiler_params=pltpu.CompilerParams(use_tc_tiling_on_sc=False),
    )
    def sc_gather(table_hbm, idx_hbm, out_hbm, idx_v, rows_v, sem):
        wid = lax.axis_index("s") * 2 + lax.axis_index("c")

        @pl.when(wid == 0)
        def _():
            pltpu.sync_copy(idx_hbm, idx_v)
            pltpu.async_copy(table_hbm.at[idx_v], rows_v, sem).wait()
            pltpu.sync_copy(rows_v, out_hbm)

    return sc_gather


_sc_gather = _make_sc_gather()


def _filter_detections_single(boxes, classification, translation, rotation):
    scoresT = classification.T
    boxesT = boxes.T
    table = jnp.concatenate(
        [boxes, rotation, translation,
         jnp.zeros((N_BOX, D_TAB - 10), jnp.float32)], axis=1)
    table = jnp.concatenate(
        [table, jnp.full((1, D_TAB), -1.0, jnp.float32)], axis=0)

    out_scores, out_labels, out_idx = pl.pallas_call(
        _fd_kernel,
        out_shape=(
            jax.ShapeDtypeStruct((1, LANES), jnp.float32),
            jax.ShapeDtypeStruct((1, LANES), jnp.int32),
            jax.ShapeDtypeStruct((1, LANES), jnp.int32),
        ),
        scratch_shapes=[pltpu.VMEM((N_CLS, N_BOX), jnp.float32),
                        pltpu.VMEM((N_CLS, N_BOX), jnp.float32),
                        pltpu.VMEM((N_CLS, N_BOX), jnp.float32),
                        pltpu.VMEM((N_CLS, N_BOX), jnp.float32),
                        pltpu.VMEM((N_CLS, N_BOX), jnp.float32),
                        pltpu.VMEM((N_CLS, N_BOX), jnp.float32),
                        pltpu.VMEM((N_CLS, N_BOX), jnp.int32)],
    )(scoresT, boxesT)

    g = _sc_gather(table, out_idx.reshape(LANES))

    b = g[:MAX_DET, 0:4]
    r = g[:MAX_DET, 4:7]
    t = g[:MAX_DET, 7:10]
    s = out_scores[0, :MAX_DET]
    l = out_labels[0, :MAX_DET]
    return b, s, l, r, t


def kernel(boxes, classification, translation, rotation):
    B = boxes.shape[0]
    obs, oss, ols, ors, ots = [], [], [], [], []
    for i in range(B):
        b, s, l, r, t = _filter_detections_single(
            boxes[i], classification[i], translation[i], rotation[i])
        obs.append(b); oss.append(s); ols.append(l); ors.append(r); ots.append(t)
    return (jnp.stack(obs), jnp.stack(oss), jnp.stack(ols),
            jnp.stack(ors), jnp.stack(ots))


# speedup vs baseline: 1.2056x; 1.2056x over previous
import functools

import jax
import jax.numpy as jnp
from jax import lax
from jax.experimental import pallas as pl
from jax.experimental.pallas import tpu as pltpu
from jax.experimental.pallas import tpu_sc as plsc

NEG_V = -1e30
SCORE_T = 0.01
NMS_T = 0.5
MAX_DET = 100
N_BOX = 20000
N_CLS = 8
LANES = 128
N_CH = 157
N_PAD = N_CH * LANES
N_ACC = 4
D_TAB = 16


def _fd_kernel(scoresT_ref, boxesT_ref,
               out_scores_ref, out_labels_ref, out_idx_ref,
               s_ref, x1_ref, y1_ref, x2_ref, y2_ref, ar_ref):
    ones = jnp.ones((N_CLS, 1), jnp.float32)
    x1_ref[:] = ones * boxesT_ref[0:1, :]
    y1_ref[:] = ones * boxesT_ref[1:2, :]
    x2_ref[:] = ones * boxesT_ref[2:3, :]
    y2_ref[:] = ones * boxesT_ref[3:4, :]
    ar_ref[:] = (x2_ref[:] - x1_ref[:]) * (y2_ref[:] - y1_ref[:])

    sc = scoresT_ref[:]
    s0 = jnp.where(sc > SCORE_T, sc, NEG_V)
    s_ref[:] = s0
    iota_p = lax.broadcasted_iota(jnp.int32, (N_CLS, N_PAD), 1)
    idx0 = jnp.argmax(s0, axis=1).reshape(N_CLS, 1)
    val0 = jnp.max(s0, axis=1, keepdims=True)
    oh0 = iota_p == idx0
    bx10 = jnp.max(jnp.where(oh0, x1_ref[:], NEG_V), axis=1, keepdims=True)
    by10 = jnp.max(jnp.where(oh0, y1_ref[:], NEG_V), axis=1, keepdims=True)
    bx20 = jnp.max(jnp.where(oh0, x2_ref[:], NEG_V), axis=1, keepdims=True)
    by20 = jnp.max(jnp.where(oh0, y2_ref[:], NEG_V), axis=1, keepdims=True)

    lane_iota = lax.broadcasted_iota(jnp.int32, (N_CLS, LANES), 1)
    minf = jnp.float32(-jnp.inf)

    def nms_body(k, carry):
        val, idx, bx1, by1, bx2, by2, vals, idxs = carry
        ba = (bx2 - bx1) * (by2 - by1)
        m = [jnp.full((N_CLS, LANES), minf, jnp.float32) for _ in range(N_ACC)]
        cj = [jnp.zeros((N_CLS, LANES), jnp.int32) for _ in range(N_ACC)]
        p1 = [jnp.zeros((N_CLS, LANES), jnp.float32) for _ in range(N_ACC)]
        q1 = [jnp.zeros((N_CLS, LANES), jnp.float32) for _ in range(N_ACC)]
        p2 = [jnp.zeros((N_CLS, LANES), jnp.float32) for _ in range(N_ACC)]
        q2 = [jnp.zeros((N_CLS, LANES), jnp.float32) for _ in range(N_ACC)]
        for j in range(N_CH):
            a = j % N_ACC
            cs = slice(j * LANES, (j + 1) * LANES)
            x1c = x1_ref[:, cs]
            y1c = y1_ref[:, cs]
            x2c = x2_ref[:, cs]
            y2c = y2_ref[:, cs]
            onehot = lane_iota == (idx - j * LANES)
            xx1 = jnp.maximum(x1c, bx1)
            yy1 = jnp.maximum(y1c, by1)
            xx2 = jnp.minimum(x2c, bx2)
            yy2 = jnp.minimum(y2c, by2)
            inter = (jnp.maximum(xx2 - xx1, 0.0)
                     * jnp.maximum(yy2 - yy1, 0.0))
            iou = inter / (ar_ref[:, cs] + ba - inter + 1e-9)
            s_new = jnp.where((iou > NMS_T) | onehot, NEG_V, s_ref[:, cs])
            s_ref[:, cs] = s_new
            upd = s_new > m[a]
            m[a] = jnp.where(upd, s_new, m[a])
            cj[a] = jnp.where(upd, j, cj[a])
            p1[a] = jnp.where(upd, x1c, p1[a])
            q1[a] = jnp.where(upd, y1c, q1[a])
            p2[a] = jnp.where(upd, x2c, p2[a])
            q2[a] = jnp.where(upd, y2c, q2[a])

        def mrg(a, b):
            ma, ja, pa1, qa1, pa2, qa2 = a
            mb, jb, pb1, qb1, pb2, qb2 = b
            take_b = (mb > ma) | ((mb == ma) & (jb < ja))
            return (jnp.where(take_b, mb, ma), jnp.where(take_b, jb, ja),
                    jnp.where(take_b, pb1, pa1), jnp.where(take_b, qb1, qa1),
                    jnp.where(take_b, pb2, pa2), jnp.where(take_b, qb2, qa2))

        accs = [(m[a], cj[a], p1[a], q1[a], p2[a], q2[a])
                for a in range(N_ACC)]
        fm, fj, f1, g1, f2, g2 = mrg(mrg(accs[0], accs[1]),
                                     mrg(accs[2], accs[3]))
        gi = fj * LANES + lane_iota
        val_n = jnp.max(fm, axis=1, keepdims=True)
        eqm = fm == val_n
        idx_n = jnp.min(jnp.where(eqm, gi, jnp.int32(2 ** 30)),
                        axis=1, keepdims=True)
        lpick = gi == idx_n
        bx1_n = jnp.max(jnp.where(lpick, f1, NEG_V), axis=1, keepdims=True)
        by1_n = jnp.max(jnp.where(lpick, g1, NEG_V), axis=1, keepdims=True)
        bx2_n = jnp.max(jnp.where(lpick, f2, NEG_V), axis=1, keepdims=True)
        by2_n = jnp.max(jnp.where(lpick, g2, NEG_V), axis=1, keepdims=True)

        here = lane_iota == k
        vals = jnp.where(here, val, vals)
        idxs = jnp.where(here, idx, idxs)
        return val_n, idx_n, bx1_n, by1_n, bx2_n, by2_n, vals, idxs

    vals0 = jnp.full((N_CLS, LANES), NEG_V, jnp.float32)
    idxs0 = jnp.zeros((N_CLS, LANES), jnp.int32)
    _, _, _, _, _, _, vals, idxs = lax.fori_loop(
        0, MAX_DET, nms_body,
        (val0, idx0, bx10, by10, bx20, by20, vals0, idxs0))

    cand = jnp.where(vals > NEG_V / 2, vals, NEG_V)
    c8 = lax.broadcasted_iota(jnp.int32, (N_CLS, 1), 0)
    lane1 = lax.broadcasted_iota(jnp.int32, (1, LANES), 1)

    def merge_body(t, carry):
        ptr, head, head_idx, tval, tlab, tidx = carry
        m = jnp.max(head, axis=(0, 1), keepdims=True)
        cw = jnp.min(jnp.where(head == m, c8, N_CLS), axis=(0, 1),
                     keepdims=True)
        isw = c8 == cw
        oidx = jnp.max(jnp.where(isw, head_idx, -1), axis=(0, 1),
                       keepdims=True)
        here = lane1 == t
        tval = jnp.where(here, m, tval)
        tlab = jnp.where(here, cw, tlab)
        tidx = jnp.where(here, oidx, tidx)
        ptr = ptr + isw.astype(jnp.int32)
        sel = lane_iota == ptr
        nh = jnp.max(jnp.where(sel, cand, NEG_V), axis=1, keepdims=True)
        nhi = jnp.max(jnp.where(sel, idxs, -1), axis=1, keepdims=True)
        head = jnp.where(isw, nh, head)
        head_idx = jnp.where(isw, nhi, head_idx)
        return ptr, head, head_idx, tval, tlab, tidx

    ptr0 = jnp.zeros((N_CLS, 1), jnp.int32)
    head0 = cand[:, 0:1]
    head_idx0 = idxs[:, 0:1]
    tval0 = jnp.full((1, LANES), NEG_V, jnp.float32)
    tlab0 = jnp.zeros((1, LANES), jnp.int32)
    tidx0 = jnp.zeros((1, LANES), jnp.int32)
    _, _, _, tval, tlab, tidx = lax.fori_loop(
        0, MAX_DET, merge_body,
        (ptr0, head0, head_idx0, tval0, tlab0, tidx0))

    valid = tval > NEG_V / 2
    out_scores_ref[:] = jnp.where(valid, tval, -1.0)
    out_labels_ref[:] = jnp.where(valid, tlab, -1)
    out_idx_ref[:] = jnp.where(valid, tidx, jnp.int32(N_BOX))


@functools.cache
def _make_sc_gather():
    mesh = plsc.VectorSubcoreMesh(core_axis_name="c", subcore_axis_name="s")

    @functools.partial(
        pl.kernel, mesh=mesh,
        out_type=jax.ShapeDtypeStruct((LANES, D_TAB), jnp.float32),
        scratch_types=[
            pltpu.VMEM((LANES,), jnp.int32),
            pltpu.VMEM((LANES, D_TAB), jnp.float32),
            pltpu.SemaphoreType.DMA,
        ],
        compiler_params=pltpu.CompilerParams(use_tc_tiling_on_sc=False),
    )
    def sc_gather(table_hbm, idx_hbm, out_hbm, idx_v, rows_v, sem):
        wid = lax.axis_index("s") * 2 + lax.axis_index("c")

        @pl.when(wid == 0)
        def _():
            pltpu.sync_copy(idx_hbm, idx_v)
            pltpu.async_copy(table_hbm.at[idx_v], rows_v, sem).wait()
            pltpu.sync_copy(rows_v, out_hbm)

    return sc_gather


def _sc_gather(table, idx):
    return _make_sc_gather()(table, idx)


def _filter_detections_single(boxes, classification, translation, rotation):
    scoresT = jnp.pad(classification.T, ((0, 0), (0, N_PAD - N_BOX)),
                      constant_values=-1.0)
    boxesT = jnp.pad(boxes.T, ((0, 0), (0, N_PAD - N_BOX)))
    table = jnp.pad(
        jnp.concatenate([boxes, rotation, translation], axis=1),
        ((0, 1), (0, D_TAB - 10)), constant_values=-1.0)

    out_scores, out_labels, out_idx = pl.pallas_call(
        _fd_kernel,
        out_shape=(
            jax.ShapeDtypeStruct((1, LANES), jnp.float32),
            jax.ShapeDtypeStruct((1, LANES), jnp.int32),
            jax.ShapeDtypeStruct((1, LANES), jnp.int32),
        ),
        scratch_shapes=[pltpu.VMEM((N_CLS, N_PAD), jnp.float32),
                        pltpu.VMEM((N_CLS, N_PAD), jnp.float32),
                        pltpu.VMEM((N_CLS, N_PAD), jnp.float32),
                        pltpu.VMEM((N_CLS, N_PAD), jnp.float32),
                        pltpu.VMEM((N_CLS, N_PAD), jnp.float32),
                        pltpu.VMEM((N_CLS, N_PAD), jnp.float32)],
    )(scoresT, boxesT)

    g = _sc_gather(table, out_idx.reshape(LANES))

    b = g[:MAX_DET, 0:4]
    r = g[:MAX_DET, 4:7]
    t = g[:MAX_DET, 7:10]
    s = out_scores[0, :MAX_DET]
    l = out_labels[0, :MAX_DET]
    return b, s, l, r, t


def kernel(boxes, classification, translation, rotation):
    B = boxes.shape[0]
    obs, oss, ols, ors, ots = [], [], [], [], []
    for i in range(B):
        b, s, l, r, t = _filter_detections_single(
            boxes[i], classification[i], translation[i], rotation[i])
        obs.append(b); oss.append(s); ols.append(l); ors.append(r); ots.append(t)
    return (jnp.stack(obs), jnp.stack(oss), jnp.stack(ols),
            jnp.stack(ors), jnp.stack(ots))


# speedup vs baseline: 1.2852x; 1.0660x over previous
import functools

import jax
import jax.numpy as jnp
from jax import lax
from jax.experimental import pallas as pl
from jax.experimental.pallas import tpu as pltpu
from jax.experimental.pallas import tpu_sc as plsc

NEG_V = -1e30
SCORE_T = 0.01
NMS_T = 0.5
MAX_DET = 100
N_BOX = 20000
N_CLS = 8
LANES = 128
N_CH = 157
N_PAD = N_CH * LANES
N_ACC = 1
D_TAB = 16


def _fd_kernel(scoresT_ref, boxesT_ref,
               out_scores_ref, out_labels_ref, out_idx_ref,
               s_ref, x1_ref, y1_ref, x2_ref, y2_ref, ar_ref):
    ones = jnp.ones((N_CLS, 1), jnp.float32)
    x1_ref[:] = ones * boxesT_ref[0:1, :]
    y1_ref[:] = ones * boxesT_ref[1:2, :]
    x2_ref[:] = ones * boxesT_ref[2:3, :]
    y2_ref[:] = ones * boxesT_ref[3:4, :]
    ar_ref[:] = (x2_ref[:] - x1_ref[:]) * (y2_ref[:] - y1_ref[:])

    sc = scoresT_ref[:]
    s0 = jnp.where(sc > SCORE_T, sc, NEG_V)
    s_ref[:] = s0
    iota_p = lax.broadcasted_iota(jnp.int32, (N_CLS, N_PAD), 1)
    idx0 = jnp.argmax(s0, axis=1).reshape(N_CLS, 1)
    val0 = jnp.max(s0, axis=1, keepdims=True)
    oh0 = iota_p == idx0
    bx10 = jnp.max(jnp.where(oh0, x1_ref[:], NEG_V), axis=1, keepdims=True)
    by10 = jnp.max(jnp.where(oh0, y1_ref[:], NEG_V), axis=1, keepdims=True)
    bx20 = jnp.max(jnp.where(oh0, x2_ref[:], NEG_V), axis=1, keepdims=True)
    by20 = jnp.max(jnp.where(oh0, y2_ref[:], NEG_V), axis=1, keepdims=True)

    lane_iota = lax.broadcasted_iota(jnp.int32, (N_CLS, LANES), 1)
    minf = jnp.float32(-jnp.inf)

    def nms_body(k, carry):
        val, idx, bx1, by1, bx2, by2, vals, idxs = carry
        ba = (bx2 - bx1) * (by2 - by1)
        m = [jnp.full((N_CLS, LANES), minf, jnp.float32) for _ in range(N_ACC)]
        cj = [jnp.zeros((N_CLS, LANES), jnp.int32) for _ in range(N_ACC)]
        p1 = [jnp.zeros((N_CLS, LANES), jnp.float32) for _ in range(N_ACC)]
        q1 = [jnp.zeros((N_CLS, LANES), jnp.float32) for _ in range(N_ACC)]
        p2 = [jnp.zeros((N_CLS, LANES), jnp.float32) for _ in range(N_ACC)]
        q2 = [jnp.zeros((N_CLS, LANES), jnp.float32) for _ in range(N_ACC)]
        for j in range(N_CH):
            a = j % N_ACC
            cs = slice(j * LANES, (j + 1) * LANES)
            x1c = x1_ref[:, cs]
            y1c = y1_ref[:, cs]
            x2c = x2_ref[:, cs]
            y2c = y2_ref[:, cs]
            onehot = lane_iota == (idx - j * LANES)
            xx1 = jnp.maximum(x1c, bx1)
            yy1 = jnp.maximum(y1c, by1)
            xx2 = jnp.minimum(x2c, bx2)
            yy2 = jnp.minimum(y2c, by2)
            inter = (jnp.maximum(xx2 - xx1, 0.0)
                     * jnp.maximum(yy2 - yy1, 0.0))
            iou = inter / (ar_ref[:, cs] + ba - inter + 1e-9)
            s_new = jnp.where((iou > NMS_T) | onehot, NEG_V, s_ref[:, cs])
            s_ref[:, cs] = s_new
            upd = s_new > m[a]
            m[a] = jnp.where(upd, s_new, m[a])
            cj[a] = jnp.where(upd, j, cj[a])
            p1[a] = jnp.where(upd, x1c, p1[a])
            q1[a] = jnp.where(upd, y1c, q1[a])
            p2[a] = jnp.where(upd, x2c, p2[a])
            q2[a] = jnp.where(upd, y2c, q2[a])

        def mrg(a, b):
            ma, ja, pa1, qa1, pa2, qa2 = a
            mb, jb, pb1, qb1, pb2, qb2 = b
            take_b = (mb > ma) | ((mb == ma) & (jb < ja))
            return (jnp.where(take_b, mb, ma), jnp.where(take_b, jb, ja),
                    jnp.where(take_b, pb1, pa1), jnp.where(take_b, qb1, qa1),
                    jnp.where(take_b, pb2, pa2), jnp.where(take_b, qb2, qa2))

        acc = (m[0], cj[0], p1[0], q1[0], p2[0], q2[0])
        for a in range(1, N_ACC):
            acc = mrg(acc, (m[a], cj[a], p1[a], q1[a], p2[a], q2[a]))
        fm, fj, f1, g1, f2, g2 = acc
        gi = fj * LANES + lane_iota
        val_n = jnp.max(fm, axis=1, keepdims=True)
        eqm = fm == val_n
        idx_n = jnp.min(jnp.where(eqm, gi, jnp.int32(2 ** 30)),
                        axis=1, keepdims=True)
        lpick = gi == idx_n
        bx1_n = jnp.max(jnp.where(lpick, f1, NEG_V), axis=1, keepdims=True)
        by1_n = jnp.max(jnp.where(lpick, g1, NEG_V), axis=1, keepdims=True)
        bx2_n = jnp.max(jnp.where(lpick, f2, NEG_V), axis=1, keepdims=True)
        by2_n = jnp.max(jnp.where(lpick, g2, NEG_V), axis=1, keepdims=True)

        here = lane_iota == k
        vals = jnp.where(here, val, vals)
        idxs = jnp.where(here, idx, idxs)
        return val_n, idx_n, bx1_n, by1_n, bx2_n, by2_n, vals, idxs

    vals0 = jnp.full((N_CLS, LANES), NEG_V, jnp.float32)
    idxs0 = jnp.zeros((N_CLS, LANES), jnp.int32)
    _, _, _, _, _, _, vals, idxs = lax.fori_loop(
        0, MAX_DET, nms_body,
        (val0, idx0, bx10, by10, bx20, by20, vals0, idxs0))

    cand = jnp.where(vals > NEG_V / 2, vals, NEG_V)
    c8 = lax.broadcasted_iota(jnp.int32, (N_CLS, 1), 0)
    lane1 = lax.broadcasted_iota(jnp.int32, (1, LANES), 1)

    def merge_body(t, carry):
        ptr, head, head_idx, tval, tlab, tidx = carry
        m = jnp.max(head, axis=(0, 1), keepdims=True)
        cw = jnp.min(jnp.where(head == m, c8, N_CLS), axis=(0, 1),
                     keepdims=True)
        isw = c8 == cw
        oidx = jnp.max(jnp.where(isw, head_idx, -1), axis=(0, 1),
                       keepdims=True)
        here = lane1 == t
        tval = jnp.where(here, m, tval)
        tlab = jnp.where(here, cw, tlab)
        tidx = jnp.where(here, oidx, tidx)
        ptr = ptr + isw.astype(jnp.int32)
        sel = lane_iota == ptr
        nh = jnp.max(jnp.where(sel, cand, NEG_V), axis=1, keepdims=True)
        nhi = jnp.max(jnp.where(sel, idxs, -1), axis=1, keepdims=True)
        head = jnp.where(isw, nh, head)
        head_idx = jnp.where(isw, nhi, head_idx)
        return ptr, head, head_idx, tval, tlab, tidx

    ptr0 = jnp.zeros((N_CLS, 1), jnp.int32)
    head0 = cand[:, 0:1]
    head_idx0 = idxs[:, 0:1]
    tval0 = jnp.full((1, LANES), NEG_V, jnp.float32)
    tlab0 = jnp.zeros((1, LANES), jnp.int32)
    tidx0 = jnp.zeros((1, LANES), jnp.int32)
    _, _, _, tval, tlab, tidx = lax.fori_loop(
        0, MAX_DET, merge_body,
        (ptr0, head0, head_idx0, tval0, tlab0, tidx0))

    valid = tval > NEG_V / 2
    out_scores_ref[:] = jnp.where(valid, tval, -1.0)
    out_labels_ref[:] = jnp.where(valid, tlab, -1)
    out_idx_ref[:] = jnp.where(valid, tidx, jnp.int32(N_BOX))


@functools.cache
def _make_sc_gather():
    mesh = plsc.VectorSubcoreMesh(core_axis_name="c", subcore_axis_name="s")

    @functools.partial(
        pl.kernel, mesh=mesh,
        out_type=jax.ShapeDtypeStruct((LANES, D_TAB), jnp.float32),
        scratch_types=[
            pltpu.VMEM((LANES,), jnp.int32),
            pltpu.VMEM((LANES, D_TAB), jnp.float32),
            pltpu.SemaphoreType.DMA,
        ],
        compiler_params=pltpu.CompilerParams(use_tc_tiling_on_sc=False),
    )
    def sc_gather(table_hbm, idx_hbm, out_hbm, idx_v, rows_v, sem):
        wid = lax.axis_index("s") * 2 + lax.axis_index("c")

        @pl.when(wid == 0)
        def _():
            pltpu.sync_copy(idx_hbm, idx_v)
            pltpu.async_copy(table_hbm.at[idx_v], rows_v, sem).wait()
            pltpu.sync_copy(rows_v, out_hbm)

    return sc_gather


def _sc_gather(table, idx):
    return _make_sc_gather()(table, idx)


def _filter_detections_single(boxes, classification, translation, rotation):
    scoresT = jnp.pad(classification.T, ((0, 0), (0, N_PAD - N_BOX)),
                      constant_values=-1.0)
    boxesT = jnp.pad(boxes.T, ((0, 0), (0, N_PAD - N_BOX)))
    table = jnp.pad(
        jnp.concatenate([boxes, rotation, translation], axis=1),
        ((0, 1), (0, D_TAB - 10)), constant_values=-1.0)

    out_scores, out_labels, out_idx = pl.pallas_call(
        _fd_kernel,
        out_shape=(
            jax.ShapeDtypeStruct((1, LANES), jnp.float32),
            jax.ShapeDtypeStruct((1, LANES), jnp.int32),
            jax.ShapeDtypeStruct((1, LANES), jnp.int32),
        ),
        scratch_shapes=[pltpu.VMEM((N_CLS, N_PAD), jnp.float32),
                        pltpu.VMEM((N_CLS, N_PAD), jnp.float32),
                        pltpu.VMEM((N_CLS, N_PAD), jnp.float32),
                        pltpu.VMEM((N_CLS, N_PAD), jnp.float32),
                        pltpu.VMEM((N_CLS, N_PAD), jnp.float32),
                        pltpu.VMEM((N_CLS, N_PAD), jnp.float32)],
    )(scoresT, boxesT)

    g = _sc_gather(table, out_idx.reshape(LANES))

    b = g[:MAX_DET, 0:4]
    r = g[:MAX_DET, 4:7]
    t = g[:MAX_DET, 7:10]
    s = out_scores[0, :MAX_DET]
    l = out_labels[0, :MAX_DET]
    return b, s, l, r, t


def kernel(boxes, classification, translation, rotation):
    B = boxes.shape[0]
    obs, oss, ols, ors, ots = [], [], [], [], []
    for i in range(B):
        b, s, l, r, t = _filter_detections_single(
            boxes[i], classification[i], translation[i], rotation[i])
        obs.append(b); oss.append(s); ols.append(l); ors.append(r); ots.append(t)
    return (jnp.stack(obs), jnp.stack(oss), jnp.stack(ols),
            jnp.stack(ors), jnp.stack(ots))
